# baseline (speedup 1.0000x reference)
import functools

import numpy as np

import jax
import jax.numpy as jnp
from jax import lax
from jax.experimental import pallas as pl
from jax.experimental.pallas import tpu as pltpu
from jax.experimental.pallas import tpu_sc as plsc

_L = 16
_NW = 32


def _sc_body(shape, kp, x_hbm, wy_hbm, par_hbm, out_hbm,
             in0, in1, ou0, ou1, tabs_v, ta_v, tb_v, par_v,
             si0, si1, so0, so1):
    b_dim, c_dim, h_dim, w_dim = shape
    planes = b_dim * c_dim
    wid = lax.axis_index("s") * 2 + lax.axis_index("c")
    pltpu.sync_copy(par_hbm, par_v)
    w_lo = par_v[pl.ds(0, _L)]
    inv_dx = par_v[pl.ds(_L, _L)]
    tmax = float(np.nextafter(np.float32(kp - 1), np.float32(0.0)))
    hh = h_dim // 2
    per_w = planes // _NW
    kpad = ta_v.shape[1]
    nvec_w = w_dim // _L

    c_base = lax.rem(wid * per_w, c_dim)
    pltpu.sync_copy(wy_hbm.at[pl.ds(c_base * kpad, per_w * kpad)], tabs_v)

    @plsc.parallel_loop(0, per_w, step=1)
    def prep_row(r):
        base = r * kpad
        for j in range(kpad // _L):
            kidx = lax.iota(jnp.int32, _L) + jnp.full((_L,), j * _L, jnp.int32)
            y0 = tabs_v[pl.ds(base + j * _L, _L)]
            idx1 = jnp.minimum(kidx + jnp.full((_L,), 1, jnp.int32),
                               jnp.full((_L,), kpad - 1, jnp.int32))
            y1 = plsc.load_gather(tabs_v, [idx1 + jnp.full((_L,), 1, jnp.int32) * base])
            dy = y1 - y0
            ta_v[r, pl.ds(j * _L, _L)] = y0 - kidx.astype(jnp.float32) * dy
            tb_v[r, pl.ds(j * _L, _L)] = dy

    def chunk_coords(k):
        p = wid * per_w + k // 2
        b = p // c_dim
        c = lax.rem(p, c_dim)
        r0 = (k % 2) * hh
        return b, c, r0

    def start_in(k, buf, sem):
        b, c, r0 = chunk_coords(k)
        return pltpu.async_copy(x_hbm.at[b, c, pl.ds(r0, hh)], buf, sem)

    def compute(k, buf, obuf):
        ta = ta_v.at[k // 2]
        tb = tb_v.at[k // 2]

        @plsc.parallel_loop(0, hh, step=1, unroll=4)
        def row_body(r):
            for j in range(nvec_w):
                x = buf[r, pl.ds(j * _L, _L)]
                tt = (x - w_lo) * inv_dx
                tt = jnp.maximum(tt, jnp.zeros((_L,), jnp.float32))
                tt = jnp.minimum(tt, jnp.full((_L,), tmax, jnp.float32))
                i0 = tt.astype(jnp.int32)
                a = plsc.load_gather(ta, [i0])
                b = plsc.load_gather(tb, [i0])
                obuf[r, pl.ds(j * _L, _L)] = a + b * tt

    def start_out(k, obuf, sem):
        b, c, r0 = chunk_coords(k)
        return pltpu.async_copy(obuf, out_hbm.at[b, c, pl.ds(r0, hh)], sem)

    def wait_out(k, obuf, sem):
        b, c, r0 = chunk_coords(k)
        pltpu.make_async_copy(obuf, out_hbm.at[b, c, pl.ds(r0, hh)], sem).wait()

    start_in(0, in0, si0)

    def pair_body(t, carry):
        ka = 2 * t
        kb = 2 * t + 1
        start_in(kb, in1, si1)
        pltpu.make_async_copy(x_hbm.at[0, 0, pl.ds(0, hh)], in0, si0).wait()

        @pl.when(t > 0)
        def _():
            wait_out(ka, ou0, so0)

        compute(ka, in0, ou0)
        start_out(ka, ou0, so0)

        @pl.when(t + 1 < per_w)
        def _():
            start_in(ka + 2, in0, si0)

        pltpu.make_async_copy(x_hbm.at[0, 0, pl.ds(0, hh)], in1, si1).wait()

        @pl.when(t > 0)
        def _():
            wait_out(kb, ou1, so1)

        compute(kb, in1, ou1)
        start_out(kb, ou1, so1)
        return carry

    lax.fori_loop(0, per_w, pair_body, 0)
    wait_out(2 * per_w - 2, ou0, so0)
    wait_out(2 * per_w - 1, ou1, so1)


def kernel(input, labels, weightx, weighty):
    B, C, H, W = input.shape
    K = weightx.shape[0]
    P = B * C
    assert P % _NW == 0 and W % _L == 0 and H % 2 == 0
    per_w = P // _NW
    assert C % per_w == 0

    wy = weighty[labels]
    Kp = ((K + _L - 1) // _L) * _L
    wy_pad = jnp.pad(wy, ((0, 0), (0, Kp - K))).reshape(-1)

    w_lo = weightx[0]
    inv_dx = (K - 1) / (weightx[-1] - w_lo)
    par = jnp.concatenate(
        [jnp.full((_L,), w_lo), jnp.full((_L,), inv_dx)]
    ).astype(jnp.float32)

    mesh = plsc.VectorSubcoreMesh(core_axis_name="c", subcore_axis_name="s")
    body = functools.partial(_sc_body, (B, C, H, W), K)
    return pl.kernel(
        body,
        mesh=mesh,
        compiler_params=pltpu.CompilerParams(needs_layout_passes=False),
        out_type=jax.ShapeDtypeStruct((B, C, H, W), jnp.float32),
        scratch_types=[
            pltpu.VMEM((H // 2, W), jnp.float32),
            pltpu.VMEM((H // 2, W), jnp.float32),
            pltpu.VMEM((H // 2, W), jnp.float32),
            pltpu.VMEM((H // 2, W), jnp.float32),
            pltpu.VMEM((P // _NW * Kp,), jnp.float32),
            pltpu.VMEM((P // _NW, Kp), jnp.float32),
            pltpu.VMEM((P // _NW, Kp), jnp.float32),
            pltpu.VMEM((2 * _L,), jnp.float32),
            pltpu.SemaphoreType.DMA,
            pltpu.SemaphoreType.DMA,
            pltpu.SemaphoreType.DMA,
            pltpu.SemaphoreType.DMA,
        ],
    )(input, wy_pad, par)


# speedup vs baseline: 1.0754x; 1.0754x over previous
import functools

import numpy as np

import jax
import jax.numpy as jnp
from jax import lax
from jax.experimental import pallas as pl
from jax.experimental.pallas import tpu as pltpu
from jax.experimental.pallas import tpu_sc as plsc

_L = 16
_NW = 32


def _sc_body(shape, kp, x_hbm, wy_hbm, par_hbm, out_hbm,
             in0, in1, ou0, ou1, tabs_v, ta_v, tb_v, par_v,
             si0, si1, so0, so1):
    b_dim, c_dim, h_dim, w_dim = shape
    planes = b_dim * c_dim
    wid = lax.axis_index("s") * 2 + lax.axis_index("c")
    pltpu.sync_copy(par_hbm, par_v)
    w_lo = par_v[pl.ds(0, _L)]
    inv_dx = par_v[pl.ds(_L, _L)]
    tmax = float(np.nextafter(np.float32(kp - 1), np.float32(0.0)))
    hh = h_dim // 2
    per_w = planes // _NW
    kpad = ta_v.shape[1]
    nvec_w = w_dim // _L

    c_base = lax.rem(wid * per_w, c_dim)
    pltpu.sync_copy(wy_hbm.at[pl.ds(c_base * kpad, per_w * kpad)], tabs_v)

    @plsc.parallel_loop(0, per_w, step=1)
    def prep_row(r):
        base = r * kpad
        for j in range(kpad // _L):
            kidx = lax.iota(jnp.int32, _L) + jnp.full((_L,), j * _L, jnp.int32)
            y0 = tabs_v[pl.ds(base + j * _L, _L)]
            idx1 = jnp.minimum(kidx + jnp.full((_L,), 1, jnp.int32),
                               jnp.full((_L,), kpad - 1, jnp.int32))
            y1 = plsc.load_gather(tabs_v, [idx1 + jnp.full((_L,), 1, jnp.int32) * base])
            dy = y1 - y0
            ta_v[r, pl.ds(j * _L, _L)] = y0 - kidx.astype(jnp.float32) * dy
            tb_v[r, pl.ds(j * _L, _L)] = dy

    def chunk_coords(k):
        p = wid * per_w + k // 2
        b = p // c_dim
        c = lax.rem(p, c_dim)
        r0 = (k % 2) * hh
        return b, c, r0

    def start_in(k, buf, sem):
        b, c, r0 = chunk_coords(k)
        return pltpu.async_copy(x_hbm.at[b, c, pl.ds(r0, hh)], buf, sem)

    def compute(k, buf, obuf):
        ta = ta_v.at[k // 2]
        tb = tb_v.at[k // 2]

        @plsc.parallel_loop(0, hh, step=1, unroll=2)
        def row_body(r):
            for j in range(nvec_w):
                x = buf[r, pl.ds(j * _L, _L)]
                tt = (x - w_lo) * inv_dx
                tt = jnp.maximum(tt, jnp.zeros((_L,), jnp.float32))
                tt = jnp.minimum(tt, jnp.full((_L,), tmax, jnp.float32))
                i0 = tt.astype(jnp.int32)
                a = plsc.load_gather(ta, [i0])
                b = plsc.load_gather(tb, [i0])
                obuf[r, pl.ds(j * _L, _L)] = a + b * tt

    def start_out(k, obuf, sem):
        b, c, r0 = chunk_coords(k)
        return pltpu.async_copy(obuf, out_hbm.at[b, c, pl.ds(r0, hh)], sem)

    def wait_out(k, obuf, sem):
        b, c, r0 = chunk_coords(k)
        pltpu.make_async_copy(obuf, out_hbm.at[b, c, pl.ds(r0, hh)], sem).wait()

    start_in(0, in0, si0)

    def pair_body(t, carry):
        ka = 2 * t
        kb = 2 * t + 1
        start_in(kb, in1, si1)
        pltpu.make_async_copy(x_hbm.at[0, 0, pl.ds(0, hh)], in0, si0).wait()

        @pl.when(t > 0)
        def _():
            wait_out(ka, ou0, so0)

        compute(ka, in0, ou0)
        start_out(ka, ou0, so0)

        @pl.when(t + 1 < per_w)
        def _():
            start_in(ka + 2, in0, si0)

        pltpu.make_async_copy(x_hbm.at[0, 0, pl.ds(0, hh)], in1, si1).wait()

        @pl.when(t > 0)
        def _():
            wait_out(kb, ou1, so1)

        compute(kb, in1, ou1)
        start_out(kb, ou1, so1)
        return carry

    lax.fori_loop(0, per_w, pair_body, 0)
    wait_out(2 * per_w - 2, ou0, so0)
    wait_out(2 * per_w - 1, ou1, so1)


def kernel(input, labels, weightx, weighty):
    B, C, H, W = input.shape
    K = weightx.shape[0]
    P = B * C
    assert P % _NW == 0 and W % _L == 0 and H % 2 == 0
    per_w = P // _NW
    assert C % per_w == 0

    wy = weighty[labels]
    Kp = ((K + _L - 1) // _L) * _L
    wy_pad = jnp.pad(wy, ((0, 0), (0, Kp - K))).reshape(-1)

    w_lo = weightx[0]
    inv_dx = (K - 1) / (weightx[-1] - w_lo)
    par = jnp.concatenate(
        [jnp.full((_L,), w_lo), jnp.full((_L,), inv_dx)]
    ).astype(jnp.float32)

    mesh = plsc.VectorSubcoreMesh(core_axis_name="c", subcore_axis_name="s")
    body = functools.partial(_sc_body, (B, C, H, W), K)
    return pl.kernel(
        body,
        mesh=mesh,
        compiler_params=pltpu.CompilerParams(
            needs_layout_passes=False, skip_device_barrier=True),
        out_type=jax.ShapeDtypeStruct((B, C, H, W), jnp.float32),
        scratch_types=[
            pltpu.VMEM((H // 2, W), jnp.float32),
            pltpu.VMEM((H // 2, W), jnp.float32),
            pltpu.VMEM((H // 2, W), jnp.float32),
            pltpu.VMEM((H // 2, W), jnp.float32),
            pltpu.VMEM((P // _NW * Kp,), jnp.float32),
            pltpu.VMEM((P // _NW, Kp), jnp.float32),
            pltpu.VMEM((P // _NW, Kp), jnp.float32),
            pltpu.VMEM((2 * _L,), jnp.float32),
            pltpu.SemaphoreType.DMA,
            pltpu.SemaphoreType.DMA,
            pltpu.SemaphoreType.DMA,
            pltpu.SemaphoreType.DMA,
        ],
    )(input, wy_pad, par)
